# baseline (speedup 1.0000x reference)
import functools

import jax
import jax.numpy as jnp
from jax import lax
from jax.experimental import pallas as pl
from jax.experimental.pallas import tpu as pltpu
from jax.experimental.pallas import tpu_sc as plsc

NC = 2
NS = 16
CH = 128


def _sc_halves(feat, src_r, dst_r, zeros, rows_pc, nch):
    n, d = feat.shape
    zrows = rows_pc // NS
    mesh = plsc.VectorSubcoreMesh(core_axis_name="c", subcore_axis_name="s")

    @functools.partial(
        pl.kernel,
        out_type=jax.ShapeDtypeStruct((NC, rows_pc, d), jnp.float32),
        mesh=mesh,
        scratch_types=[
            pltpu.VMEM((nch, CH), jnp.int32),
            pltpu.VMEM((nch, CH), jnp.int32),
            pltpu.VMEM((3, CH, d), jnp.float32),
            pltpu.VMEM_SHARED((rows_pc, d), jnp.float32),
            [pltpu.SemaphoreType.DMA] * 3,
        ],
    )
    def run(feat_hbm, src_hbm, dst_hbm, zero_hbm, out_hbm,
            isrc, idst, rows, agg, gsems):
        c = lax.axis_index("c")
        s = lax.axis_index("s")

        pltpu.sync_copy(zero_hbm, agg.at[pl.ds(s * zrows, zrows)])
        pltpu.sync_copy(src_hbm.at[s], isrc)
        pltpu.sync_copy(dst_hbm.at[c, s], idst)
        plsc.subcore_barrier()

        for b in range(3):
            pltpu.async_copy(feat_hbm.at[isrc.at[b]], rows.at[b], gsems[b])

        def body(jj, carry):
            for b in range(3):
                j = jj * 3 + b
                pltpu.make_async_copy(
                    feat_hbm.at[isrc.at[j]], rows.at[b], gsems[b]).wait()
                pltpu.sync_copy(rows.at[b], agg.at[idst.at[j]], add=True)
                nxt = j + 3

                @pl.when(nxt < nch)
                def _():
                    pltpu.async_copy(
                        feat_hbm.at[isrc.at[nxt]], rows.at[b], gsems[b])
            return carry

        lax.fori_loop(0, nch // 3, body, 0)
        plsc.subcore_barrier()
        pltpu.sync_copy(agg.at[pl.ds(s * zrows, zrows)],
                        out_hbm.at[c, pl.ds(s * zrows, zrows)])

    return run(feat, src_r, dst_r, zeros)


def _tc_mlp(x, parts, wa_t, ba, g, be, wb_t, bb, half, final):
    n, d = x.shape

    def body(x_ref, p_ref, wa_ref, ba_ref, g_ref, be_ref, wb_ref, bb_ref,
             o_ref):
        agg = jnp.concatenate([p_ref[0, :half], p_ref[1, :half]], axis=0)
        h = x_ref[...] + agg
        z = jnp.dot(h, wa_ref[...], preferred_element_type=jnp.float32)
        z = z + ba_ref[...]
        mean = jnp.mean(z, axis=0, keepdims=True)
        var = jnp.mean(jnp.square(z - mean), axis=0, keepdims=True)
        zn = (z - mean) / jnp.sqrt(var + 1e-5) * g_ref[...] + be_ref[...]
        zn = jnp.maximum(zn, 0.0)
        out = jnp.dot(zn, wb_ref[...], preferred_element_type=jnp.float32)
        out = out + bb_ref[...]
        if final:
            m = jnp.max(out, axis=-1, keepdims=True)
            e = out - m
            out = e - jnp.log(jnp.sum(jnp.exp(e), axis=-1, keepdims=True))
        else:
            out = jnp.maximum(out, 0.0)
        o_ref[...] = out

    return pl.pallas_call(
        body,
        out_shape=jax.ShapeDtypeStruct((n, d), jnp.float32),
    )(x, parts, wa_t, ba, g, be, wb_t, bb)


def kernel(x, edge_index, W1a, b1a, g1, be1, W1b, b1b,
           W2a, b2a, g2, be2, W2b, b2b):
    n, d = x.shape
    e = edge_index.shape[1]
    assert n % (2 * NC) == 0
    half = n // NC

    zrows = -(-(half + 1) // NS)
    zrows = -(-zrows // 8) * 8
    rows_pc = zrows * NS
    sink = half

    nch = -(-(-(-e // (NS * CH))) // 3) * 3
    e_pad = NS * nch * CH
    pad = e_pad - e

    src = edge_index[0].astype(jnp.int32)
    dst = edge_index[1].astype(jnp.int32)
    src_r = jnp.concatenate(
        [src, jnp.zeros((pad,), jnp.int32)]).reshape(NS, nch, CH)
    base = jnp.arange(NC, dtype=jnp.int32)[:, None] * half
    in_half = (dst[None, :] >= base) & (dst[None, :] < base + half)
    dst_c = jnp.where(in_half, dst[None, :] - base, sink)
    dst_r = jnp.concatenate(
        [dst_c, jnp.full((NC, pad), sink, jnp.int32)],
        axis=1).reshape(NC, NS, nch, CH)
    zeros = jnp.zeros((zrows, d), jnp.float32)

    def prep(wa, ba_, gg, bee, wb, bb_):
        return (wa.T, ba_.reshape(1, -1), gg.reshape(1, -1),
                bee.reshape(1, -1), wb.T, bb_.reshape(1, -1))

    p1 = _sc_halves(x, src_r, dst_r, zeros, rows_pc, nch)
    t1 = _tc_mlp(x, p1, *prep(W1a, b1a, g1, be1, W1b, b1b),
                 half=half, final=False)
    p2 = _sc_halves(t1, src_r, dst_r, zeros, rows_pc, nch)
    return _tc_mlp(t1, p2, *prep(W2a, b2a, g2, be2, W2b, b2b),
                   half=half, final=True)


# speedup vs baseline: 1.3535x; 1.3535x over previous
import functools

import jax
import jax.numpy as jnp
from jax import lax
from jax.experimental import pallas as pl
from jax.experimental.pallas import tpu as pltpu
from jax.experimental.pallas import tpu_sc as plsc

NC = 2
NS = 16
CH = 128
NPH = 2


def _sc_partials(feat, src_r, dst_r, zeros, n_pad, nch):
    n, d = feat.shape
    zrows = n_pad // NS
    pch = nch // NPH
    mesh = plsc.VectorSubcoreMesh(core_axis_name="c", subcore_axis_name="s")

    @functools.partial(
        pl.kernel,
        out_type=jax.ShapeDtypeStruct((NC, n_pad, d), jnp.float32),
        mesh=mesh,
        scratch_types=[
            pltpu.VMEM((pch, CH), jnp.int32),
            pltpu.VMEM((pch, CH), jnp.int32),
            pltpu.VMEM((2, CH, d), jnp.float32),
            pltpu.VMEM_SHARED((n_pad, d), jnp.float32),
            pltpu.SemaphoreType.DMA,
            pltpu.SemaphoreType.DMA,
        ],
    )
    def run(feat_hbm, src_hbm, dst_hbm, zero_hbm, out_hbm,
            isrc, idst, rows, agg, sem0, sem1):
        c = lax.axis_index("c")
        s = lax.axis_index("s")
        w = c * NS + s
        sems = (sem0, sem1)

        pltpu.sync_copy(zero_hbm, agg.at[pl.ds(s * zrows, zrows)])
        plsc.subcore_barrier()

        for ph in range(NPH):
            pltpu.sync_copy(src_hbm.at[w, pl.ds(ph * pch, pch)], isrc)
            pltpu.sync_copy(dst_hbm.at[w, pl.ds(ph * pch, pch)], idst)

            for b in range(2):
                pltpu.async_copy(
                    feat_hbm.at[isrc.at[b]], rows.at[b], sems[b])

            def body(jj, carry):
                for b in range(2):
                    j = jj * 2 + b
                    pltpu.make_async_copy(
                        feat_hbm.at[isrc.at[j]], rows.at[b],
                        sems[b]).wait()
                    pltpu.sync_copy(
                        rows.at[b], agg.at[idst.at[j]], add=True)
                    nxt = j + 2

                    @pl.when(nxt < pch)
                    def _():
                        pltpu.async_copy(
                            feat_hbm.at[isrc.at[nxt]], rows.at[b],
                            sems[b])
                return carry

            lax.fori_loop(0, pch // 2, body, 0)

        plsc.subcore_barrier()
        pltpu.sync_copy(agg.at[pl.ds(s * zrows, zrows)],
                        out_hbm.at[c, pl.ds(s * zrows, zrows)])

    return run(feat, src_r, dst_r, zeros)


def _tc_mlp(x, parts, wa_t, ba, g, be, wb_t, bb, final):
    n, d = x.shape

    def body(x_ref, p_ref, wa_ref, ba_ref, g_ref, be_ref, wb_ref, bb_ref,
             o_ref):
        h = x_ref[...] + p_ref[0, :n] + p_ref[1, :n]
        z = jnp.dot(h, wa_ref[...], preferred_element_type=jnp.float32)
        z = z + ba_ref[...]
        mean = jnp.mean(z, axis=0, keepdims=True)
        var = jnp.mean(jnp.square(z - mean), axis=0, keepdims=True)
        zn = (z - mean) / jnp.sqrt(var + 1e-5) * g_ref[...] + be_ref[...]
        zn = jnp.maximum(zn, 0.0)
        out = jnp.dot(zn, wb_ref[...], preferred_element_type=jnp.float32)
        out = out + bb_ref[...]
        if final:
            m = jnp.max(out, axis=-1, keepdims=True)
            e = out - m
            out = e - jnp.log(jnp.sum(jnp.exp(e), axis=-1, keepdims=True))
        else:
            out = jnp.maximum(out, 0.0)
        o_ref[...] = out

    return pl.pallas_call(
        body,
        out_shape=jax.ShapeDtypeStruct((n, d), jnp.float32),
    )(x, parts, wa_t, ba, g, be, wb_t, bb)


def kernel(x, edge_index, W1a, b1a, g1, be1, W1b, b1b,
           W2a, b2a, g2, be2, W2b, b2b):
    n, d = x.shape
    e = edge_index.shape[1]
    nw = NC * NS
    assert n % NS == 0

    zrows = -(-(n + 1) // NS)
    zrows = -(-zrows // 8) * 8
    n_pad = zrows * NS
    sink = n

    nch = -(-(-(-e // (nw * CH))) // (2 * NPH)) * (2 * NPH)
    e_pad = nw * nch * CH
    pad = e_pad - e

    src = edge_index[0].astype(jnp.int32)
    dst = edge_index[1].astype(jnp.int32)
    src_r = jnp.concatenate(
        [src, jnp.zeros((pad,), jnp.int32)]).reshape(nw, nch, CH)
    dst_r = jnp.concatenate(
        [dst, jnp.full((pad,), sink, jnp.int32)]).reshape(nw, nch, CH)
    zeros = jnp.zeros((zrows, d), jnp.float32)

    def prep(wa, ba_, gg, bee, wb, bb_):
        return (wa.T, ba_.reshape(1, -1), gg.reshape(1, -1),
                bee.reshape(1, -1), wb.T, bb_.reshape(1, -1))

    p1 = _sc_partials(x, src_r, dst_r, zeros, n_pad, nch)
    t1 = _tc_mlp(x, p1, *prep(W1a, b1a, g1, be1, W1b, b1b), final=False)
    p2 = _sc_partials(t1, src_r, dst_r, zeros, n_pad, nch)
    return _tc_mlp(t1, p2, *prep(W2a, b2a, g2, be2, W2b, b2b), final=True)
